# baseline (speedup 1.0000x reference)
import jax
import jax.numpy as jnp
from jax import lax
from jax.experimental import pallas as pl
from jax.experimental.pallas import tpu as pltpu
from jax.experimental.pallas import tpu_sc as plsc

N_NODES = 10000
N_EDGES = 320000
D = 128
R = 8

NC = 1
NS = 16
NW = NC * NS

CHUNK = 80
RB = 25
NBLK = N_EDGES // (CHUNK * RB * NW)
N_PAD = 10240
NODES_PER_TILE = N_PAD // NS
NODES_LAST = N_NODES - (NS - 1) * NODES_PER_TILE
NBUF = 4


def _mm_body(feat_ref, w_ref, out_ref):
    out_ref[0] = jnp.dot(feat_ref[...], w_ref[0],
                         preferred_element_type=jnp.float32)


def _typed_transform(feat, W):
    BN = 1000
    NB = N_NODES // BN
    return pl.pallas_call(
        _mm_body,
        grid=(NB, R),
        in_specs=[
            pl.BlockSpec((BN, D), lambda n, r: (n, 0)),
            pl.BlockSpec((1, D, D), lambda n, r: (r, 0, 0)),
        ],
        out_specs=pl.BlockSpec((1, BN, D), lambda n, r: (r, n, 0)),
        out_shape=jax.ShapeDtypeStruct((R, N_NODES, D), jnp.float32),
    )(feat, W)


def _sc_body(table, feat, gidx4, dst4, out, gidx_v, dst_v,
             rows_a, rows_b, rows_c, rows_d, shared_acc,
             gsem_a, gsem_b, gsem_c, gsem_d, ssem_a, ssem_b, ssem_c, ssem_d):
    s = lax.axis_index("s")
    base = s * NODES_PER_TILE
    bufs = (rows_a, rows_b, rows_c, rows_d)
    gsems = (gsem_a, gsem_b, gsem_c, gsem_d)
    ssems = (ssem_a, ssem_b, ssem_c, ssem_d)

    @pl.when(s < NS - 1)
    def _():
        pltpu.sync_copy(feat.at[pl.ds(base, NODES_PER_TILE)],
                        shared_acc.at[pl.ds(base, NODES_PER_TILE)])

    @pl.when(s == NS - 1)
    def _():
        pltpu.sync_copy(feat.at[pl.ds(base, NODES_LAST)],
                        shared_acc.at[pl.ds(base, NODES_LAST)])

    plsc.subcore_barrier()

    def _block(b, _):
        pltpu.sync_copy(gidx4.at[s, b], gidx_v)
        pltpu.sync_copy(dst4.at[s, b], dst_v)

        g = [None] * RB
        sc = [None] * RB
        for r in range(RB):
            if r >= NBUF:
                sc[r - NBUF].wait()
            g[r] = pltpu.async_copy(table.at[gidx_v.at[r]],
                                    bufs[r % NBUF], gsems[r % NBUF])
            if r >= 1:
                g[r - 1].wait()
                sc[r - 1] = pltpu.async_copy(
                    bufs[(r - 1) % NBUF],
                    shared_acc.at[dst_v.at[r - 1]],
                    ssems[(r - 1) % NBUF], add=True)
        g[RB - 1].wait()
        sc[RB - 1] = pltpu.async_copy(
            bufs[(RB - 1) % NBUF],
            shared_acc.at[dst_v.at[RB - 1]],
            ssems[(RB - 1) % NBUF], add=True)
        for r in range(RB - NBUF, RB):
            sc[r].wait()
        return 0
    lax.fori_loop(0, NBLK, _block, 0)

    plsc.subcore_barrier()

    @pl.when(s < NS - 1)
    def _():
        pltpu.sync_copy(shared_acc.at[pl.ds(base, NODES_PER_TILE)],
                        out.at[pl.ds(base, NODES_PER_TILE)])

    @pl.when(s == NS - 1)
    def _():
        pltpu.sync_copy(shared_acc.at[pl.ds(base, NODES_LAST)],
                        out.at[pl.ds(base, NODES_LAST)])


def _sc_scatter(table2d, feat, gidx4, dst4):
    mesh = plsc.VectorSubcoreMesh(core_axis_name="c", subcore_axis_name="s",
                                  num_cores=NC)
    return pl.kernel(
        _sc_body,
        out_type=jax.ShapeDtypeStruct((N_NODES, D), jnp.float32),
        mesh=mesh,
        scratch_types=[
            pltpu.VMEM((RB, CHUNK), jnp.int32),
            pltpu.VMEM((RB, CHUNK), jnp.int32),
            pltpu.VMEM((CHUNK, D), jnp.float32),
            pltpu.VMEM((CHUNK, D), jnp.float32),
            pltpu.VMEM((CHUNK, D), jnp.float32),
            pltpu.VMEM((CHUNK, D), jnp.float32),
            pltpu.VMEM_SHARED((N_PAD, D), jnp.float32),
            pltpu.SemaphoreType.DMA,
            pltpu.SemaphoreType.DMA,
            pltpu.SemaphoreType.DMA,
            pltpu.SemaphoreType.DMA,
            pltpu.SemaphoreType.DMA,
            pltpu.SemaphoreType.DMA,
            pltpu.SemaphoreType.DMA,
            pltpu.SemaphoreType.DMA,
        ],
    )(table2d, feat, gidx4, dst4)


@jax.jit
def kernel(feat, edge_index, etypes, W):
    table = _typed_transform(feat, W).reshape(R * N_NODES, D)
    gidx4 = (etypes.astype(jnp.int32) * N_NODES
             + edge_index[0]).reshape(NW, NBLK, RB, CHUNK)
    dst4 = edge_index[1].reshape(NW, NBLK, RB, CHUNK)
    return _sc_scatter(table, feat, gidx4, dst4)


# speedup vs baseline: 1.1277x; 1.1277x over previous
import jax
import jax.numpy as jnp
from jax import lax
from jax.experimental import pallas as pl
from jax.experimental.pallas import tpu as pltpu
from jax.experimental.pallas import tpu_sc as plsc

N_NODES = 10000
N_EDGES = 320000
D = 128
R = 8

NC = 1
NS = 16
NW = NC * NS

CHUNK = 80
RB = 25
NBLK = N_EDGES // (CHUNK * RB * NW)
N_PAD = 10240
NODES_PER_TILE = N_PAD // NS
NODES_LAST = N_NODES - (NS - 1) * NODES_PER_TILE
NBUF = 3


def _mm_body(feat_ref, w_ref, out_ref):
    out_ref[0] = jnp.dot(feat_ref[...], w_ref[0],
                         preferred_element_type=jnp.float32)


def _typed_transform(feat, W):
    BN = 1000
    NB = N_NODES // BN
    return pl.pallas_call(
        _mm_body,
        grid=(NB, R),
        in_specs=[
            pl.BlockSpec((BN, D), lambda n, r: (n, 0)),
            pl.BlockSpec((1, D, D), lambda n, r: (r, 0, 0)),
        ],
        out_specs=pl.BlockSpec((1, BN, D), lambda n, r: (r, n, 0)),
        out_shape=jax.ShapeDtypeStruct((R, N_NODES, D), jnp.float32),
    )(feat, W)


def _sc_body(table, feat, gidx4, dst4, out, gidx_v0, dst_v0, gidx_v1, dst_v1,
             rows_a, rows_b, rows_c, shared_acc,
             gsem_a, gsem_b, gsem_c, ssem_a, ssem_b, ssem_c, isem_a, isem_b):
    s = lax.axis_index("s")
    base = s * NODES_PER_TILE
    bufs = (rows_a, rows_b, rows_c)
    gsems = (gsem_a, gsem_b, gsem_c)
    ssems = (ssem_a, ssem_b, ssem_c)
    idx_sets = ((gidx_v0, dst_v0), (gidx_v1, dst_v1))
    isems = (isem_a, isem_b)

    @pl.when(s < NS - 1)
    def _():
        pltpu.sync_copy(feat.at[pl.ds(base, NODES_PER_TILE)],
                        shared_acc.at[pl.ds(base, NODES_PER_TILE)])

    @pl.when(s == NS - 1)
    def _():
        pltpu.sync_copy(feat.at[pl.ds(base, NODES_LAST)],
                        shared_acc.at[pl.ds(base, NODES_LAST)])

    plsc.subcore_barrier()

    pltpu.sync_copy(gidx4.at[s, 0], idx_sets[0][0])
    pltpu.sync_copy(dst4.at[s, 0], idx_sets[0][1])
    for b in range(NBLK):
        gidx_v, dst_v = idx_sets[b % 2]
        ipf = [None, None]
        if b + 1 < NBLK:
            ngidx, ndst = idx_sets[(b + 1) % 2]
            ipf[0] = pltpu.async_copy(gidx4.at[s, b + 1], ngidx,
                                      isems[(b + 1) % 2])
            ipf[1] = pltpu.async_copy(dst4.at[s, b + 1], ndst,
                                      isems[(b + 1) % 2])
        g = [None] * RB
        sc = [None] * RB
        for r in range(RB):
            if r >= NBUF:
                sc[r - NBUF].wait()
            g[r] = pltpu.async_copy(table.at[gidx_v.at[r]],
                                    bufs[r % NBUF], gsems[r % NBUF])
            if r >= 1:
                g[r - 1].wait()
                sc[r - 1] = pltpu.async_copy(
                    bufs[(r - 1) % NBUF],
                    shared_acc.at[dst_v.at[r - 1]],
                    ssems[(r - 1) % NBUF], add=True)
        g[RB - 1].wait()
        sc[RB - 1] = pltpu.async_copy(
            bufs[(RB - 1) % NBUF],
            shared_acc.at[dst_v.at[RB - 1]],
            ssems[(RB - 1) % NBUF], add=True)
        for r in range(RB - NBUF, RB):
            sc[r].wait()
        if b + 1 < NBLK:
            ipf[0].wait()
            ipf[1].wait()

    plsc.subcore_barrier()

    @pl.when(s < NS - 1)
    def _():
        pltpu.sync_copy(shared_acc.at[pl.ds(base, NODES_PER_TILE)],
                        out.at[pl.ds(base, NODES_PER_TILE)])

    @pl.when(s == NS - 1)
    def _():
        pltpu.sync_copy(shared_acc.at[pl.ds(base, NODES_LAST)],
                        out.at[pl.ds(base, NODES_LAST)])


def _sc_scatter(table2d, feat, gidx4, dst4):
    mesh = plsc.VectorSubcoreMesh(core_axis_name="c", subcore_axis_name="s",
                                  num_cores=NC)
    return pl.kernel(
        _sc_body,
        out_type=jax.ShapeDtypeStruct((N_NODES, D), jnp.float32),
        mesh=mesh,
        scratch_types=[
            pltpu.VMEM((RB, CHUNK), jnp.int32),
            pltpu.VMEM((RB, CHUNK), jnp.int32),
            pltpu.VMEM((RB, CHUNK), jnp.int32),
            pltpu.VMEM((RB, CHUNK), jnp.int32),
            pltpu.VMEM((CHUNK, D), jnp.float32),
            pltpu.VMEM((CHUNK, D), jnp.float32),
            pltpu.VMEM((CHUNK, D), jnp.float32),
            pltpu.VMEM_SHARED((N_PAD, D), jnp.float32),
            pltpu.SemaphoreType.DMA,
            pltpu.SemaphoreType.DMA,
            pltpu.SemaphoreType.DMA,
            pltpu.SemaphoreType.DMA,
            pltpu.SemaphoreType.DMA,
            pltpu.SemaphoreType.DMA,
            pltpu.SemaphoreType.DMA,
            pltpu.SemaphoreType.DMA,
        ],
    )(table2d, feat, gidx4, dst4)


@jax.jit
def kernel(feat, edge_index, etypes, W):
    table = _typed_transform(feat, W).reshape(R * N_NODES, D)
    gidx4 = (etypes.astype(jnp.int32) * N_NODES
             + edge_index[0]).reshape(NW, NBLK, RB, CHUNK)
    dst4 = edge_index[1].reshape(NW, NBLK, RB, CHUNK)
    return _sc_scatter(table, feat, gidx4, dst4)


# speedup vs baseline: 1.1393x; 1.0103x over previous
import jax
import jax.numpy as jnp
from jax import lax
from jax.experimental import pallas as pl
from jax.experimental.pallas import tpu as pltpu
from jax.experimental.pallas import tpu_sc as plsc

N_NODES = 10000
N_EDGES = 320000
D = 128
R = 8

NC = 1
NS = 16
NW = NC * NS

CHUNK = 80
RB = 50
NBLK = N_EDGES // (CHUNK * RB * NW)
N_PAD = 10240
NODES_PER_TILE = N_PAD // NS
NODES_LAST = N_NODES - (NS - 1) * NODES_PER_TILE
NBUF = 3


def _mm_body(feat_ref, w_ref, out_ref):
    out_ref[0] = jnp.dot(feat_ref[...], w_ref[0],
                         preferred_element_type=jnp.float32)


def _typed_transform(feat, W):
    BN = 1000
    NB = N_NODES // BN
    return pl.pallas_call(
        _mm_body,
        grid=(NB, R),
        in_specs=[
            pl.BlockSpec((BN, D), lambda n, r: (n, 0)),
            pl.BlockSpec((1, D, D), lambda n, r: (r, 0, 0)),
        ],
        out_specs=pl.BlockSpec((1, BN, D), lambda n, r: (r, n, 0)),
        out_shape=jax.ShapeDtypeStruct((R, N_NODES, D), jnp.float32),
    )(feat, W)


def _sc_body(table, feat, gidx4, dst4, out, gidx_v, dst_v,
             rows_a, rows_b, rows_c, shared_acc,
             gsem_a, gsem_b, gsem_c, ssem_a, ssem_b, ssem_c):
    s = lax.axis_index("s")
    base = s * NODES_PER_TILE
    bufs = (rows_a, rows_b, rows_c)
    gsems = (gsem_a, gsem_b, gsem_c)
    ssems = (ssem_a, ssem_b, ssem_c)

    @pl.when(s < NS - 1)
    def _():
        pltpu.sync_copy(feat.at[pl.ds(base, NODES_PER_TILE)],
                        shared_acc.at[pl.ds(base, NODES_PER_TILE)])

    @pl.when(s == NS - 1)
    def _():
        pltpu.sync_copy(feat.at[pl.ds(base, NODES_LAST)],
                        shared_acc.at[pl.ds(base, NODES_LAST)])

    plsc.subcore_barrier()

    def _block(b, _):
        pltpu.sync_copy(gidx4.at[s, b], gidx_v)
        pltpu.sync_copy(dst4.at[s, b], dst_v)

        g = [None] * RB
        sc = [None] * RB
        for r in range(RB):
            if r >= NBUF:
                sc[r - NBUF].wait()
            g[r] = pltpu.async_copy(table.at[gidx_v.at[r]],
                                    bufs[r % NBUF], gsems[r % NBUF])
            if r >= 1:
                g[r - 1].wait()
                sc[r - 1] = pltpu.async_copy(
                    bufs[(r - 1) % NBUF],
                    shared_acc.at[dst_v.at[r - 1]],
                    ssems[(r - 1) % NBUF], add=True)
        g[RB - 1].wait()
        sc[RB - 1] = pltpu.async_copy(
            bufs[(RB - 1) % NBUF],
            shared_acc.at[dst_v.at[RB - 1]],
            ssems[(RB - 1) % NBUF], add=True)
        for r in range(RB - NBUF, RB):
            sc[r].wait()
        return 0
    lax.fori_loop(0, NBLK, _block, 0)

    plsc.subcore_barrier()

    @pl.when(s < NS - 1)
    def _():
        pltpu.sync_copy(shared_acc.at[pl.ds(base, NODES_PER_TILE)],
                        out.at[pl.ds(base, NODES_PER_TILE)])

    @pl.when(s == NS - 1)
    def _():
        pltpu.sync_copy(shared_acc.at[pl.ds(base, NODES_LAST)],
                        out.at[pl.ds(base, NODES_LAST)])


def _sc_scatter(table2d, feat, gidx4, dst4):
    mesh = plsc.VectorSubcoreMesh(core_axis_name="c", subcore_axis_name="s",
                                  num_cores=NC)
    return pl.kernel(
        _sc_body,
        out_type=jax.ShapeDtypeStruct((N_NODES, D), jnp.float32),
        mesh=mesh,
        scratch_types=[
            pltpu.VMEM((RB, CHUNK), jnp.int32),
            pltpu.VMEM((RB, CHUNK), jnp.int32),
            pltpu.VMEM((CHUNK, D), jnp.float32),
            pltpu.VMEM((CHUNK, D), jnp.float32),
            pltpu.VMEM((CHUNK, D), jnp.float32),
            pltpu.VMEM_SHARED((N_PAD, D), jnp.float32),
            pltpu.SemaphoreType.DMA,
            pltpu.SemaphoreType.DMA,
            pltpu.SemaphoreType.DMA,
            pltpu.SemaphoreType.DMA,
            pltpu.SemaphoreType.DMA,
            pltpu.SemaphoreType.DMA,
        ],
    )(table2d, feat, gidx4, dst4)


@jax.jit
def kernel(feat, edge_index, etypes, W):
    table = _typed_transform(feat, W).reshape(R * N_NODES, D)
    gidx4 = (etypes.astype(jnp.int32) * N_NODES
             + edge_index[0]).reshape(NW, NBLK, RB, CHUNK)
    dst4 = edge_index[1].reshape(NW, NBLK, RB, CHUNK)
    return _sc_scatter(table, feat, gidx4, dst4)
